# baseline (speedup 1.0000x reference)
import functools

import numpy as np
import jax
import jax.numpy as jnp
from jax import lax
from jax.experimental import pallas as pl
from jax.experimental.pallas import tpu as pltpu
from jax.experimental.pallas import tpu_sc as plsc

D = 64
S = 200
B = 4096
NC, NS, L = 2, 16, 16
NW = NC * NS
SEQ_PER_W = B // NW
C = 2
CHUNK_ROWS = C * S
N_CHUNK = SEQ_PER_W // C


def _pe_table():
    position = jnp.arange(S, dtype=jnp.float32)[:, None]
    div_term = jnp.exp(
        jnp.arange(0, D, 2, dtype=jnp.float32) * (-np.log(10000.0) / D)
    )
    pe = jnp.zeros((S, D), jnp.float32)
    pe = pe.at[:, 0::2].set(jnp.sin(position * div_term))
    pe = pe.at[:, 1::2].set(jnp.cos(position * div_term))
    return pe


_mesh = plsc.VectorSubcoreMesh(core_axis_name="c", subcore_axis_name="s")


@functools.partial(
    pl.kernel,
    out_type=jax.ShapeDtypeStruct((B * S, D), jnp.float32),
    mesh=_mesh,
    compiler_params=pltpu.CompilerParams(use_tc_tiling_on_sc=False),
    scratch_types=[
        pltpu.VMEM((CHUNK_ROWS,), jnp.int32),
        pltpu.VMEM((CHUNK_ROWS, D), jnp.float32),
        pltpu.VMEM((S, D), jnp.float32),
        pltpu.SemaphoreType.DMA,
    ],
)
def _emb_kernel(x_hbm, table_hbm, pe_hbm, out_hbm, idx_v, rows_v, pe_v, sem):
    wid = lax.axis_index("s") * NC + lax.axis_index("c")
    pltpu.sync_copy(pe_hbm, pe_v)
    base_row = wid * SEQ_PER_W * S

    def chunk_body(it, carry):
        row0 = base_row + it * CHUNK_ROWS
        pltpu.sync_copy(x_hbm.at[pl.ds(row0, CHUNK_ROWS)], idx_v)
        pltpu.async_copy(table_hbm.at[idx_v], rows_v, sem).wait()

        def add_body(r, c2):
            for j in range(D // L):
                pe_j = pe_v[r, pl.ds(j * L, L)]
                for c in range(C):
                    rr = c * S + r
                    rows_v[rr, pl.ds(j * L, L)] = (
                        rows_v[rr, pl.ds(j * L, L)] + pe_j
                    )
            return c2

        lax.fori_loop(0, S, add_body, 0)
        pltpu.sync_copy(rows_v, out_hbm.at[pl.ds(row0, CHUNK_ROWS)])
        return carry

    lax.fori_loop(0, N_CHUNK, chunk_body, 0)


def kernel(x, table):
    pe = _pe_table()
    out = _emb_kernel(x.reshape(-1), table, pe)
    return out.reshape(B, S, D)
